# baseline (speedup 1.0000x reference)
import jax
import jax.numpy as jnp
from jax.experimental import pallas as pl
from jax.experimental.pallas import tpu as pltpu

H = 1024
E = 8
TM = 128
T = 4096


def _moe_body(xf_ref, br_ref, wg_ref, bg_ref, *rest):
    wb_refs = rest[:E]
    out_ref = rest[E]
    wn_scr = rest[E + 1]

    @pl.when(pl.program_id(0) == 0)
    def _stage_w():
        rows = jax.lax.broadcasted_iota(jnp.int32, (H, 512), 0)
        cols = jax.lax.broadcasted_iota(jnp.int32, (H, 512), 1)
        for h in range(2):
            ident_h = (rows == cols + h * 512).astype(jnp.bfloat16)
            for e in range(E):
                half = jax.lax.dot_general(
                    wb_refs[e][...], ident_h, (((0,), (0,)), ((), ())),
                    preferred_element_type=jnp.float32,
                )
                wn_scr[e, :, h * 512:(h + 1) * 512] = half.astype(jnp.bfloat16)

    xf = xf_ref[...]
    xb = xf.astype(jnp.bfloat16)

    logits = jax.lax.dot_general(
        xf, wg_ref[...], (((1,), (0,)), ((), ())),
        preferred_element_type=jnp.float32,
    ) + bg_ref[...]
    idx = jax.lax.broadcasted_iota(jnp.int32, (TM, E), 1)
    m1 = jnp.max(logits, axis=-1, keepdims=True)
    i1 = jnp.min(jnp.where(logits == m1, idx, E), axis=-1, keepdims=True)
    mask1 = idx == i1
    l2 = jnp.where(mask1, jnp.finfo(jnp.float32).min, logits)
    m2 = jnp.max(l2, axis=-1, keepdims=True)
    i2 = jnp.min(jnp.where(l2 == m2, idx, E), axis=-1, keepdims=True)
    mask2 = idx == i2
    tt = jnp.exp(m2 - m1)
    w1 = 1.0 / (1.0 + tt)
    w = jnp.where(mask1, w1, 0.0) + jnp.where(mask2, 1.0 - w1, 0.0)

    acc = jax.lax.dot_general(w, br_ref[...], (((1,), (0,)), ((), ())))
    for e in range(E):
        ye = jax.lax.dot_general(
            xb, wn_scr[e], (((1,), (0,)), ((), ())),
            preferred_element_type=jnp.float32,
        )
        acc = acc + w[:, e:e + 1] * ye
    out_ref[...] = acc


def kernel(x, W_experts, b_experts, W_gate, b_gate):
    B, S, _ = x.shape
    xf = x.reshape(T, H)
    wb = W_experts.astype(jnp.bfloat16).reshape(H, E, 1, H)
    br = b_experts.reshape(H, E).T
    wg = W_gate.T
    bg = b_gate.reshape(1, E)

    def w_spec(e):
        return pl.BlockSpec((H, None, None, H), lambda t, _e=e: (0, _e, 0, 0))

    out = pl.pallas_call(
        _moe_body,
        grid=(T // TM,),
        in_specs=[
            pl.BlockSpec((TM, H), lambda t: (t, 0)),
            pl.BlockSpec((E, H), lambda t: (0, 0)),
            pl.BlockSpec((H, E), lambda t: (0, 0)),
            pl.BlockSpec((1, E), lambda t: (0, 0)),
        ] + [w_spec(e) for e in range(E)],
        out_specs=pl.BlockSpec((TM, H), lambda t: (t, 0)),
        out_shape=jax.ShapeDtypeStruct((T, H), jnp.float32),
        scratch_shapes=[pltpu.VMEM((E, H, H), jnp.bfloat16)],
    )(xf, br, wg, bg, *[wb] * E)
    return out.reshape(B, S, H)


# speedup vs baseline: 2.3287x; 2.3287x over previous
import jax
import jax.numpy as jnp
from jax.experimental import pallas as pl

H = 1024
E = 8
TM = 256
T = 4096


def _moe_body(xf_ref, wt_ref, br_ref, wg_ref, bg_ref, out_ref):
    xf = xf_ref[...]
    xb = xf.astype(jnp.bfloat16)

    logits = jax.lax.dot_general(
        xf, wg_ref[...], (((1,), (0,)), ((), ())),
        preferred_element_type=jnp.float32,
    ) + bg_ref[...]
    idx = jax.lax.broadcasted_iota(jnp.int32, (TM, E), 1)
    m1 = jnp.max(logits, axis=-1, keepdims=True)
    i1 = jnp.min(jnp.where(logits == m1, idx, E), axis=-1, keepdims=True)
    mask1 = idx == i1
    l2 = jnp.where(mask1, jnp.finfo(jnp.float32).min, logits)
    m2 = jnp.max(l2, axis=-1, keepdims=True)
    i2 = jnp.min(jnp.where(l2 == m2, idx, E), axis=-1, keepdims=True)
    mask2 = idx == i2
    tt = jnp.exp(m2 - m1)
    w1 = 1.0 / (1.0 + tt)
    w = jnp.where(mask1, w1, 0.0) + jnp.where(mask2, 1.0 - w1, 0.0)

    acc = jax.lax.dot_general(w, br_ref[...], (((1,), (0,)), ((), ())))

    for e in range(E):
        ye = jax.lax.dot_general(
            xb, wt_ref[e], (((1,), (0,)), ((), ())),
            preferred_element_type=jnp.float32,
        )
        acc = acc + w[:, e:e + 1] * ye
    out_ref[...] = acc


def kernel(x, W_experts, b_experts, W_gate, b_gate):
    B, S, _ = x.shape
    xf = x.reshape(T, H)
    wt = W_experts.astype(jnp.bfloat16).reshape(H, E, H).transpose(1, 2, 0)
    br = b_experts.reshape(H, E).T
    wg = W_gate.T
    bg = b_gate.reshape(1, E)

    out = pl.pallas_call(
        _moe_body,
        grid=(T // TM,),
        in_specs=[
            pl.BlockSpec((TM, H), lambda i: (i, 0)),
            pl.BlockSpec((E, H, H), lambda i: (0, 0, 0)),
            pl.BlockSpec((E, H), lambda i: (0, 0)),
            pl.BlockSpec((H, E), lambda i: (0, 0)),
            pl.BlockSpec((1, E), lambda i: (0, 0)),
        ],
        out_specs=pl.BlockSpec((TM, H), lambda i: (i, 0)),
        out_shape=jax.ShapeDtypeStruct((T, H), jnp.float32),
    )(xf, wt, br, wg, bg)
    return out.reshape(B, S, H)


# speedup vs baseline: 2.4299x; 1.0434x over previous
import jax
import jax.numpy as jnp
from jax.experimental import pallas as pl

H = 1024
E = 8
TM = 512
T = 4096


def _moe_body(xf_ref, wt_ref, br_ref, wg_ref, bg_ref, out_ref):
    xf = xf_ref[...]
    xb = xf.astype(jnp.bfloat16)

    logits = jax.lax.dot_general(
        xf, wg_ref[...], (((1,), (0,)), ((), ())),
        preferred_element_type=jnp.float32,
    ) + bg_ref[...]
    idx = jax.lax.broadcasted_iota(jnp.int32, (TM, E), 1)
    m1 = jnp.max(logits, axis=-1, keepdims=True)
    i1 = jnp.min(jnp.where(logits == m1, idx, E), axis=-1, keepdims=True)
    mask1 = idx == i1
    l2 = jnp.where(mask1, jnp.finfo(jnp.float32).min, logits)
    m2 = jnp.max(l2, axis=-1, keepdims=True)
    i2 = jnp.min(jnp.where(l2 == m2, idx, E), axis=-1, keepdims=True)
    mask2 = idx == i2
    tt = jnp.exp(m2 - m1)
    w1 = 1.0 / (1.0 + tt)
    w = jnp.where(mask1, w1, 0.0) + jnp.where(mask2, 1.0 - w1, 0.0)

    acc = jax.lax.dot_general(w, br_ref[...], (((1,), (0,)), ((), ())))

    for e in range(E):
        ye = jax.lax.dot_general(
            xb, wt_ref[e], (((1,), (0,)), ((), ())),
            preferred_element_type=jnp.float32,
        )
        acc = acc + w[:, e:e + 1] * ye
    out_ref[...] = acc


def kernel(x, W_experts, b_experts, W_gate, b_gate):
    B, S, _ = x.shape
    xf = x.reshape(T, H)
    wt = W_experts.astype(jnp.bfloat16).reshape(H, E, H).transpose(1, 2, 0)
    br = b_experts.reshape(H, E).T
    wg = W_gate.T
    bg = b_gate.reshape(1, E)

    out = pl.pallas_call(
        _moe_body,
        grid=(T // TM,),
        in_specs=[
            pl.BlockSpec((TM, H), lambda i: (i, 0)),
            pl.BlockSpec((E, H, H), lambda i: (0, 0, 0)),
            pl.BlockSpec((E, H), lambda i: (0, 0)),
            pl.BlockSpec((H, E), lambda i: (0, 0)),
            pl.BlockSpec((1, E), lambda i: (0, 0)),
        ],
        out_specs=pl.BlockSpec((TM, H), lambda i: (i, 0)),
        out_shape=jax.ShapeDtypeStruct((T, H), jnp.float32),
    )(xf, wt, br, wg, bg)
    return out.reshape(B, S, H)


# speedup vs baseline: 2.4396x; 1.0040x over previous
import jax
import jax.numpy as jnp
from jax.experimental import pallas as pl

H = 1024
E = 8
TM = 1024
T = 4096


def _moe_body(xf_ref, wt_ref, br_ref, wg_ref, bg_ref, out_ref):
    xf = xf_ref[...]
    xb = xf.astype(jnp.bfloat16)

    logits = jax.lax.dot_general(
        xf, wg_ref[...], (((1,), (0,)), ((), ())),
        preferred_element_type=jnp.float32,
    ) + bg_ref[...]
    idx = jax.lax.broadcasted_iota(jnp.int32, (TM, E), 1)
    m1 = jnp.max(logits, axis=-1, keepdims=True)
    i1 = jnp.min(jnp.where(logits == m1, idx, E), axis=-1, keepdims=True)
    mask1 = idx == i1
    l2 = jnp.where(mask1, jnp.finfo(jnp.float32).min, logits)
    m2 = jnp.max(l2, axis=-1, keepdims=True)
    i2 = jnp.min(jnp.where(l2 == m2, idx, E), axis=-1, keepdims=True)
    mask2 = idx == i2
    tt = jnp.exp(m2 - m1)
    w1 = 1.0 / (1.0 + tt)
    w = jnp.where(mask1, w1, 0.0) + jnp.where(mask2, 1.0 - w1, 0.0)

    acc = jax.lax.dot_general(w, br_ref[...], (((1,), (0,)), ((), ())))

    for e in range(E):
        ye = jax.lax.dot_general(
            xb, wt_ref[e], (((1,), (0,)), ((), ())),
            preferred_element_type=jnp.float32,
        )
        acc = acc + w[:, e:e + 1] * ye
    out_ref[...] = acc


def kernel(x, W_experts, b_experts, W_gate, b_gate):
    B, S, _ = x.shape
    xf = x.reshape(T, H)
    wt = W_experts.astype(jnp.bfloat16).reshape(H, E, H).transpose(1, 2, 0)
    br = b_experts.reshape(H, E).T
    wg = W_gate.T
    bg = b_gate.reshape(1, E)

    out = pl.pallas_call(
        _moe_body,
        grid=(T // TM,),
        in_specs=[
            pl.BlockSpec((TM, H), lambda i: (i, 0)),
            pl.BlockSpec((E, H, H), lambda i: (0, 0, 0)),
            pl.BlockSpec((E, H), lambda i: (0, 0)),
            pl.BlockSpec((H, E), lambda i: (0, 0)),
            pl.BlockSpec((1, E), lambda i: (0, 0)),
        ],
        out_specs=pl.BlockSpec((TM, H), lambda i: (i, 0)),
        out_shape=jax.ShapeDtypeStruct((T, H), jnp.float32),
    )(xf, wt, br, wg, bg)
    return out.reshape(B, S, H)
